# baseline (speedup 1.0000x reference)
import jax
import jax.numpy as jnp
from jax.experimental import pallas as pl

_B, _S = 8, 4
_T = _B * _S
_HIDDEN = 512
_INTER = 1024
_E = 8
_K = 2
_IH = _INTER // 2


def _moe_kernel(x_ref, rw_ref, w13g_ref, w13u_ref, w2a_ref, w2b_ref, out_ref):
    e = pl.program_id(0)
    x = x_ref[...]

    logits = jnp.dot(x, rw_ref[...].T,
                     preferred_element_type=jnp.float32)
    m = jnp.max(logits, axis=-1, keepdims=True)
    ex = jnp.exp(logits - m)
    probs = ex / jnp.sum(ex, axis=-1, keepdims=True)

    cols = jax.lax.broadcasted_iota(jnp.int32, probs.shape, 1)
    i1 = jnp.argmax(probs, axis=-1, keepdims=True)
    v1 = jnp.max(probs, axis=-1)
    masked = jnp.where(cols == i1, -1.0, probs)
    i2 = jnp.argmax(masked, axis=-1, keepdims=True)
    v2 = jnp.max(masked, axis=-1)
    denom = v1 + v2 + 1e-6
    sel = (cols == i1) | (cols == i2)
    dense_w = jnp.where(sel, probs, 0.0) / denom[:, None]
    w_e = jnp.sum(jnp.where(cols == e, dense_w, 0.0), axis=-1)

    gate = jnp.dot(x, w13g_ref[0], preferred_element_type=jnp.float32)
    up = jnp.dot(x, w13u_ref[0], preferred_element_type=jnp.float32)
    h = (gate * jax.nn.sigmoid(gate)) * up
    out_e = (jnp.dot(h[:, :_IH], w2a_ref[0], preferred_element_type=jnp.float32)
             + jnp.dot(h[:, _IH:], w2b_ref[0], preferred_element_type=jnp.float32))

    contrib = out_e * w_e[:, None]

    @pl.when(e == 0)
    def _():
        out_ref[...] = contrib

    @pl.when(e != 0)
    def _():
        out_ref[...] = out_ref[...] + contrib


@jax.jit
def kernel(x, router_w, w13, w2):
    xt = x.reshape(_T, _HIDDEN)
    out = pl.pallas_call(
        _moe_kernel,
        grid=(_E,),
        in_specs=[
            pl.BlockSpec((_T, _HIDDEN), lambda e: (0, 0)),
            pl.BlockSpec((_E, _HIDDEN), lambda e: (0, 0)),
            pl.BlockSpec((1, _HIDDEN, _INTER), lambda e: (e, 0, 0)),
            pl.BlockSpec((1, _HIDDEN, _INTER), lambda e: (e, 0, 1)),
            pl.BlockSpec((1, _IH, _HIDDEN), lambda e: (e, 0, 0)),
            pl.BlockSpec((1, _IH, _HIDDEN), lambda e: (e, 1, 0)),
        ],
        out_specs=pl.BlockSpec((_T, _HIDDEN), lambda e: (0, 0)),
        out_shape=jax.ShapeDtypeStruct((_T, _HIDDEN), jnp.float32),
    )(xt, router_w, w13, w13, w2, w2)
    return out.reshape(_B, _S, _HIDDEN)


# speedup vs baseline: 1.0178x; 1.0178x over previous
import jax
import jax.numpy as jnp
from jax.experimental import pallas as pl

_B, _S = 8, 4
_T = _B * _S
_HIDDEN = 512
_INTER = 1024
_E = 8
_K = 2
_EPB = 2
_G = _E // _EPB


def _moe_kernel(x_ref, rw_ref, w13_ref, w2_ref, out_ref):
    g = pl.program_id(0)
    x = x_ref[...]

    logits = jnp.dot(x, rw_ref[...].T,
                     preferred_element_type=jnp.float32)
    m = jnp.max(logits, axis=-1, keepdims=True)
    ex = jnp.exp(logits - m)
    probs = ex / jnp.sum(ex, axis=-1, keepdims=True)

    cols = jax.lax.broadcasted_iota(jnp.int32, probs.shape, 1)
    i1 = jnp.argmax(probs, axis=-1, keepdims=True)
    v1 = jnp.max(probs, axis=-1)
    masked = jnp.where(cols == i1, -1.0, probs)
    i2 = jnp.argmax(masked, axis=-1, keepdims=True)
    v2 = jnp.max(masked, axis=-1)
    denom = v1 + v2 + 1e-6
    sel = (cols == i1) | (cols == i2)
    dense_w = jnp.where(sel, probs, 0.0) / denom[:, None]

    acc = jnp.zeros((_T, _HIDDEN), jnp.float32)
    for s in range(_EPB):
        e = g * _EPB + s
        w_e = jnp.sum(jnp.where(cols == e, dense_w, 0.0), axis=-1)
        h13 = jnp.dot(x, w13_ref[s], preferred_element_type=jnp.float32)
        gate = h13[:, :_INTER]
        up = h13[:, _INTER:]
        h = (gate * jax.nn.sigmoid(gate)) * up
        out_e = jnp.dot(h, w2_ref[s], preferred_element_type=jnp.float32)
        acc = acc + out_e * w_e[:, None]

    @pl.when(g == 0)
    def _():
        out_ref[...] = acc

    @pl.when(g != 0)
    def _():
        out_ref[...] = out_ref[...] + acc


@jax.jit
def kernel(x, router_w, w13, w2):
    xt = x.reshape(_T, _HIDDEN)
    out = pl.pallas_call(
        _moe_kernel,
        grid=(_G,),
        in_specs=[
            pl.BlockSpec((_T, _HIDDEN), lambda g: (0, 0)),
            pl.BlockSpec((_E, _HIDDEN), lambda g: (0, 0)),
            pl.BlockSpec((_EPB, _HIDDEN, 2 * _INTER), lambda g: (g, 0, 0)),
            pl.BlockSpec((_EPB, _INTER, _HIDDEN), lambda g: (g, 0, 0)),
        ],
        out_specs=pl.BlockSpec((_T, _HIDDEN), lambda g: (0, 0)),
        out_shape=jax.ShapeDtypeStruct((_T, _HIDDEN), jnp.float32),
    )(xt, router_w, w13, w2)
    return out.reshape(_B, _S, _HIDDEN)
